# baseline (speedup 1.0000x reference)
import jax
import jax.numpy as jnp
import numpy as np
from jax.experimental import pallas as pl
from jax.experimental.pallas import tpu as pltpu

IN_DIM = 256
OUT_DIM = 128
KNN = 10
B = 16
N = 512


def _layernorm(y, g, be):
    mu = jnp.mean(y, axis=-1, keepdims=True)
    var = jnp.var(y, axis=-1, keepdims=True)
    return (y - mu) / jnp.sqrt(var + 1e-5) * g + be


def _lna(x, p):
    return jax.nn.selu(_layernorm(x @ p["W"] + p["b"], p["g"], p["be"]))


def _gat(x, p):
    q = x @ p["Wq"]
    k = x @ p["Wk"]
    v = x @ p["Wv"]
    a = jax.nn.softmax(q @ jnp.swapaxes(k, -1, -2) / np.sqrt(q.shape[-1]), axis=-1)
    return a @ v + x @ p["Wr"]


def _mm(a, b):
    return jax.lax.dot_general(a, b, (((1,), (0,)), ((), ())),
                               preferred_element_type=jnp.float32)


def _conv_max_body(e_ref, w_ref, b_ref, h_ref):
    val = _mm(e_ref[0], w_ref[...]) + b_ref[...]
    k = pl.program_id(1)

    @pl.when(k == 0)
    def _init():
        h_ref[0] = val

    @pl.when(k > 0)
    def _fold():
        h_ref[0] = jnp.maximum(h_ref[0], val)


def _conv_max(e, W, bias):
    d2 = e.shape[-1]
    dout = W.shape[1]
    ekm = jnp.transpose(e, (0, 2, 1, 3)).reshape(B, KNN * N, d2)
    return pl.pallas_call(
        _conv_max_body,
        grid=(B, KNN),
        in_specs=[
            pl.BlockSpec((1, N, d2), lambda b, k: (b, k, 0)),
            pl.BlockSpec((d2, dout), lambda b, k: (0, 0)),
            pl.BlockSpec((1, dout), lambda b, k: (0, 0)),
        ],
        out_specs=pl.BlockSpec((1, N, dout), lambda b, k: (b, 0, 0)),
        out_shape=jax.ShapeDtypeStruct((B, N, dout), jnp.float32),
        compiler_params=pltpu.CompilerParams(
            dimension_semantics=("arbitrary", "arbitrary"),
            vmem_limit_bytes=100 * 1024 * 1024,
        ),
    )(ekm, W, bias.reshape(1, -1))


def _xo(x, p, use_pallas):
    g = _gat(x, p["gat"])
    xn = g / (jnp.linalg.norm(g, axis=-1, keepdims=True) + 1e-9)
    sim = xn @ jnp.swapaxes(xn, -1, -2)
    sim = sim - 2.0 * jnp.eye(N, dtype=x.dtype)
    _, idx = jax.lax.top_k(sim, KNN)
    xj = jax.vmap(lambda xb, ib: xb[ib])(g, idx)
    xi = jnp.broadcast_to(g[:, :, None, :], xj.shape)
    e = jnp.concatenate([xj - xi, xi], axis=-1)
    cp = p["conv"]
    if use_pallas:
        h = _conv_max(e, cp["W"], cp["b"])
    else:
        h = jnp.max(e @ cp["W"] + cp["b"], axis=2)
    return jax.nn.selu(_layernorm(h, cp["g"], cp["be"]))


def _lin(x, p):
    return x @ p["W"] + p["b"]


def kernel(obj_encs, n_nodes, params):
    b = n_nodes.shape[0]
    n = obj_encs.shape[0] // b
    x = obj_encs.reshape(b, n, obj_encs.shape[-1])
    x = _lna(x, params["f_p1"])
    x1 = _xo(x, params["xo1"], use_pallas=False)
    x2 = _xo(_lna(x1, params["f_p2"]), params["xo2"], use_pallas=False)
    x3 = _xo(_lna(x2, params["f_p3"]), params["xo3"], use_pallas=True)
    cat = jnp.concatenate([x, _lna(x1, params["p1"]), _lna(x2, params["p2"]),
                           _lna(x3, params["p3"])], axis=-1)
    out = _lin(_lin(cat, params["ph1"]), params["ph2"])
    out = out / (jnp.linalg.norm(out, axis=-1, keepdims=True) + 1e-9)
    return out.reshape(b * n, -1)


# speedup vs baseline: 1.2130x; 1.2130x over previous
import jax
import jax.numpy as jnp
import numpy as np
from jax.experimental import pallas as pl
from jax.experimental.pallas import tpu as pltpu

IN_DIM = 256
OUT_DIM = 128
KNN = 10
B = 16
N = 512


def _layernorm(y, g, be):
    mu = jnp.mean(y, axis=-1, keepdims=True)
    var = jnp.var(y, axis=-1, keepdims=True)
    return (y - mu) / jnp.sqrt(var + 1e-5) * g + be


def _lna(x, p):
    return jax.nn.selu(_layernorm(x @ p["W"] + p["b"], p["g"], p["be"]))


def _gat(x, p):
    q = x @ p["Wq"]
    k = x @ p["Wk"]
    v = x @ p["Wv"]
    a = jax.nn.softmax(q @ jnp.swapaxes(k, -1, -2) / np.sqrt(q.shape[-1]), axis=-1)
    return a @ v + x @ p["Wr"]


def _mm(a, b):
    return jax.lax.dot_general(a, b, (((1,), (0,)), ((), ())),
                               preferred_element_type=jnp.float32)


def _conv_max_body(e_ref, w_ref, b_ref, h_ref):
    val = _mm(e_ref[0], w_ref[...]) + b_ref[...]
    k = pl.program_id(1)

    @pl.when(k == 0)
    def _init():
        h_ref[0] = val

    @pl.when(k > 0)
    def _fold():
        h_ref[0] = jnp.maximum(h_ref[0], val)


def _conv_max(e, W, bias):
    d2 = e.shape[-1]
    dout = W.shape[1]
    ekm = jnp.transpose(e, (0, 2, 1, 3)).reshape(B, KNN * N, d2)
    return pl.pallas_call(
        _conv_max_body,
        grid=(B, KNN),
        in_specs=[
            pl.BlockSpec((1, N, d2), lambda b, k: (b, k, 0)),
            pl.BlockSpec((d2, dout), lambda b, k: (0, 0)),
            pl.BlockSpec((1, dout), lambda b, k: (0, 0)),
        ],
        out_specs=pl.BlockSpec((1, N, dout), lambda b, k: (b, 0, 0)),
        out_shape=jax.ShapeDtypeStruct((B, N, dout), jnp.float32),
        compiler_params=pltpu.CompilerParams(
            dimension_semantics=("arbitrary", "arbitrary"),
            vmem_limit_bytes=100 * 1024 * 1024,
        ),
    )(ekm, W, bias.reshape(1, -1))


def _mm_hi(a, b):
    return jax.lax.dot_general(a, b, (((1,), (0,)), ((), ())),
                               preferred_element_type=jnp.float32,
                               precision=jax.lax.Precision.HIGHEST)


def _knn_conv_body(idx_ref, x_ref, w_ref, b_ref, h_ref):
    idx = idx_ref[0]
    x = x_ref[0]
    W = w_ref[...]
    bias = b_ref[...]
    n = x.shape[0]
    cols = jax.lax.broadcasted_iota(jnp.int32, (n, n), 1)
    agg = jnp.full((n, W.shape[1]), -1e30, jnp.float32)
    for k in range(KNN):
        oh = cols == idx[:, k][:, None]
        xj = _mm_hi(oh.astype(jnp.float32), x)
        e = jnp.concatenate([xj - x, x], axis=-1)
        agg = jnp.maximum(agg, _mm(e, W) + bias)
    h_ref[0] = agg


def _knn_conv(idx, x, W, bias):
    dh = x.shape[-1]
    dout = W.shape[1]
    return pl.pallas_call(
        _knn_conv_body,
        grid=(B,),
        in_specs=[
            pl.BlockSpec((1, N, KNN), lambda i: (i, 0, 0)),
            pl.BlockSpec((1, N, dh), lambda i: (i, 0, 0)),
            pl.BlockSpec((2 * dh, dout), lambda i: (0, 0)),
            pl.BlockSpec((1, dout), lambda i: (0, 0)),
        ],
        out_specs=pl.BlockSpec((1, N, dout), lambda i: (i, 0, 0)),
        out_shape=jax.ShapeDtypeStruct((B, N, dout), jnp.float32),
        compiler_params=pltpu.CompilerParams(
            dimension_semantics=("arbitrary",),
            vmem_limit_bytes=100 * 1024 * 1024,
        ),
    )(idx, x, W, bias.reshape(1, -1))


def _xo(x, p, use_pallas):
    g = _gat(x, p["gat"])
    xn = g / (jnp.linalg.norm(g, axis=-1, keepdims=True) + 1e-9)
    sim = xn @ jnp.swapaxes(xn, -1, -2)
    sim = sim - 2.0 * jnp.eye(N, dtype=x.dtype)
    _, idx = jax.lax.top_k(sim, KNN)
    cp = p["conv"]
    if use_pallas:
        h = _knn_conv(idx, g, cp["W"], cp["b"])
    else:
        xj = jax.vmap(lambda xb, ib: xb[ib])(g, idx)
        xi = jnp.broadcast_to(g[:, :, None, :], xj.shape)
        e = jnp.concatenate([xj - xi, xi], axis=-1)
        h = jnp.max(e @ cp["W"] + cp["b"], axis=2)
    return jax.nn.selu(_layernorm(h, cp["g"], cp["be"]))


def _lin(x, p):
    return x @ p["W"] + p["b"]


def kernel(obj_encs, n_nodes, params):
    b = n_nodes.shape[0]
    n = obj_encs.shape[0] // b
    x = obj_encs.reshape(b, n, obj_encs.shape[-1])
    x = _lna(x, params["f_p1"])
    x1 = _xo(x, params["xo1"], use_pallas=False)
    x2 = _xo(_lna(x1, params["f_p2"]), params["xo2"], use_pallas=False)
    x3 = _xo(_lna(x2, params["f_p3"]), params["xo3"], use_pallas=True)
    cat = jnp.concatenate([x, _lna(x1, params["p1"]), _lna(x2, params["p2"]),
                           _lna(x3, params["p3"])], axis=-1)
    out = _lin(_lin(cat, params["ph1"]), params["ph2"])
    out = out / (jnp.linalg.norm(out, axis=-1, keepdims=True) + 1e-9)
    return out.reshape(b * n, -1)


# speedup vs baseline: 1.2132x; 1.0001x over previous
import jax
import jax.numpy as jnp
import numpy as np
from jax.experimental import pallas as pl
from jax.experimental.pallas import tpu as pltpu

IN_DIM = 256
OUT_DIM = 128
KNN = 10
B = 16
N = 512


def _layernorm(y, g, be):
    mu = jnp.mean(y, axis=-1, keepdims=True)
    var = jnp.var(y, axis=-1, keepdims=True)
    return (y - mu) / jnp.sqrt(var + 1e-5) * g + be


def _lna(x, p):
    return jax.nn.selu(_layernorm(x @ p["W"] + p["b"], p["g"], p["be"]))


def _gat(x, p):
    q = x @ p["Wq"]
    k = x @ p["Wk"]
    v = x @ p["Wv"]
    a = jax.nn.softmax(q @ jnp.swapaxes(k, -1, -2) / np.sqrt(q.shape[-1]), axis=-1)
    return a @ v + x @ p["Wr"]


def _mm(a, b):
    return jax.lax.dot_general(a, b, (((1,), (0,)), ((), ())),
                               preferred_element_type=jnp.float32)


def _mm_hi(a, b):
    return jax.lax.dot_general(a, b, (((1,), (0,)), ((), ())),
                               preferred_element_type=jnp.float32,
                               precision=jax.lax.Precision.HIGHEST)


def _knn_conv_body(idx_ref, x_ref, w_ref, b_ref, h_ref):
    idx = idx_ref[0]
    x = x_ref[0]
    W = w_ref[...]
    bias = b_ref[...]
    n = x.shape[0]
    cols = jax.lax.broadcasted_iota(jnp.int32, (n, n), 1)
    agg = jnp.full((n, W.shape[1]), -1e30, jnp.float32)
    for k in range(KNN):
        oh = cols == idx[:, k][:, None]
        xj = _mm_hi(oh.astype(jnp.float32), x)
        e = jnp.concatenate([xj - x, x], axis=-1)
        agg = jnp.maximum(agg, _mm(e, W) + bias)
    h_ref[0] = agg


def _knn_conv(idx, x, W, bias):
    dh = x.shape[-1]
    dout = W.shape[1]
    return pl.pallas_call(
        _knn_conv_body,
        grid=(B,),
        in_specs=[
            pl.BlockSpec((1, N, KNN), lambda i: (i, 0, 0)),
            pl.BlockSpec((1, N, dh), lambda i: (i, 0, 0)),
            pl.BlockSpec((2 * dh, dout), lambda i: (0, 0)),
            pl.BlockSpec((1, dout), lambda i: (0, 0)),
        ],
        out_specs=pl.BlockSpec((1, N, dout), lambda i: (i, 0, 0)),
        out_shape=jax.ShapeDtypeStruct((B, N, dout), jnp.float32),
        compiler_params=pltpu.CompilerParams(
            dimension_semantics=("arbitrary",),
            vmem_limit_bytes=100 * 1024 * 1024,
        ),
    )(idx, x, W, bias.reshape(1, -1))


def _xo(x, p, use_pallas):
    g = _gat(x, p["gat"])
    xn = g / (jnp.linalg.norm(g, axis=-1, keepdims=True) + 1e-9)
    sim = xn @ jnp.swapaxes(xn, -1, -2)
    sim = sim - 2.0 * jnp.eye(N, dtype=x.dtype)
    _, idx = jax.lax.top_k(sim, KNN)
    cp = p["conv"]
    if use_pallas:
        h = _knn_conv(idx, g, cp["W"], cp["b"])
    else:
        xj = jax.vmap(lambda xb, ib: xb[ib])(g, idx)
        xi = jnp.broadcast_to(g[:, :, None, :], xj.shape)
        e = jnp.concatenate([xj - xi, xi], axis=-1)
        h = jnp.max(e @ cp["W"] + cp["b"], axis=2)
    return jax.nn.selu(_layernorm(h, cp["g"], cp["be"]))


def _lin(x, p):
    return x @ p["W"] + p["b"]


def kernel(obj_encs, n_nodes, params):
    b = n_nodes.shape[0]
    n = obj_encs.shape[0] // b
    x = obj_encs.reshape(b, n, obj_encs.shape[-1])
    x = _lna(x, params["f_p1"])
    x1 = _xo(x, params["xo1"], use_pallas=False)
    x2 = _xo(_lna(x1, params["f_p2"]), params["xo2"], use_pallas=False)
    x3 = _xo(_lna(x2, params["f_p3"]), params["xo3"], use_pallas=True)
    cat = jnp.concatenate([x, _lna(x1, params["p1"]), _lna(x2, params["p2"]),
                           _lna(x3, params["p3"])], axis=-1)
    out = _lin(_lin(cat, params["ph1"]), params["ph2"])
    out = out / (jnp.linalg.norm(out, axis=-1, keepdims=True) + 1e-9)
    return out.reshape(b * n, -1)
